# baseline (speedup 1.0000x reference)
import functools

import jax
import jax.numpy as jnp
from jax import lax
from jax.experimental import pallas as pl
from jax.experimental.pallas import tpu as pltpu
from jax.experimental.pallas import tpu_sc as plsc

_NC, _NS = 2, 16
_NW = _NC * _NS
_NBUF = 4


def kernel(x, emb):
    b, s, d = x.shape
    assert b == _NBUF
    pe = emb[:s]
    rows_w = s // _NW
    ch_rows = 16
    n_ch = rows_w // ch_rows

    mesh = plsc.VectorSubcoreMesh(core_axis_name="c", subcore_axis_name="s")

    scratch = (
        [pltpu.VMEM((ch_rows, d), jnp.float32) for _ in range(2)]
        + [pltpu.VMEM((ch_rows, d), jnp.float32) for _ in range(_NBUF)]
        + [pltpu.SemaphoreType.DMA for _ in range(2 + 2 * _NBUF)]
    )

    @functools.partial(
        pl.kernel,
        out_type=jax.ShapeDtypeStruct((b, s, d), jnp.float32),
        mesh=mesh,
        scratch_types=scratch,
    )
    def sc_add(x_hbm, emb_hbm, out_hbm, *bufs):
        ev = bufs[0:2]
        xv = bufs[2:2 + _NBUF]
        esem = bufs[2 + _NBUF:4 + _NBUF]
        xsem = bufs[4 + _NBUF:4 + 2 * _NBUF]
        osem = bufs[4 + 2 * _NBUF:4 + 3 * _NBUF]

        wid = lax.axis_index("s") * _NC + lax.axis_index("c")
        base = wid * rows_w

        def row(c):
            return base + c * ch_rows

        def load(c, j):
            pltpu.async_copy(x_hbm.at[j, pl.ds(row(c), ch_rows)], xv[j], xsem[j])

        def load_wait(c, j):
            pltpu.make_async_copy(
                x_hbm.at[j, pl.ds(row(c), ch_rows)], xv[j], xsem[j]
            ).wait()

        def store(c, j):
            pltpu.async_copy(xv[j], out_hbm.at[j, pl.ds(row(c), ch_rows)], osem[j])

        def store_wait(c, j):
            pltpu.make_async_copy(
                xv[j], out_hbm.at[j, pl.ds(row(c), ch_rows)], osem[j]
            ).wait()

        pltpu.async_copy(emb_hbm.at[pl.ds(row(0), ch_rows)], ev[0], esem[0])
        pltpu.async_copy(emb_hbm.at[pl.ds(row(1), ch_rows)], ev[1], esem[1])
        load(0, 0)
        load(0, 1)

        @pl.loop(0, n_ch, step=2)
        def _chunks(c0):
            for u in range(8):
                cc, j = divmod(u, 4)
                c = c0 + cc
                load_wait(c, j)
                if j == 0:
                    pltpu.make_async_copy(
                        emb_hbm.at[pl.ds(row(c), ch_rows)], ev[cc], esem[cc]
                    ).wait()

                @plsc.parallel_loop(0, d, step=16)
                def _vec(o):
                    for r in range(ch_rows):
                        xv[j][r, pl.ds(o, 16)] = (
                            xv[j][r, pl.ds(o, 16)] + ev[cc][r, pl.ds(o, 16)]
                        )

                store(c, j)
                cb, jb = divmod(u - 2, 4)
                cf, jf = divmod(u + 2, 4)
                if u < 2:
                    @pl.when(c0 > 0)
                    def _():
                        store_wait(c0 + cb, jb)
                    load(c0 + cf, jf)
                elif u < 6:
                    store_wait(c0 + cb, jb)
                    load(c0 + cf, jf)
                else:
                    store_wait(c0 + cb, jb)

                    @pl.when(c0 + 2 < n_ch)
                    def _():
                        load(c0 + cf, jf)
                if j == 3:
                    @pl.when(c0 + 2 + cc < n_ch)
                    def _():
                        pltpu.async_copy(
                            emb_hbm.at[pl.ds(row(c0 + 2 + cc), ch_rows)],
                            ev[cc], esem[cc],
                        )

        store_wait(n_ch - 1, 2)
        store_wait(n_ch - 1, 3)

    return sc_add(x, pe)


# speedup vs baseline: 1.0799x; 1.0799x over previous
import functools

import jax
import jax.numpy as jnp
from jax import lax
from jax.experimental import pallas as pl
from jax.experimental.pallas import tpu as pltpu
from jax.experimental.pallas import tpu_sc as plsc

_NC, _NS = 2, 16
_NW = _NC * _NS


def kernel(x, emb):
    b, s, d = x.shape
    pe = emb[:s]
    rows_w = s // _NW
    ch_rows = 8
    n_ch = rows_w // ch_rows

    mesh = plsc.VectorSubcoreMesh(core_axis_name="c", subcore_axis_name="s")

    scratch = (
        [pltpu.VMEM((ch_rows, d), jnp.float32) for _ in range(2)]
        + [pltpu.VMEM((ch_rows, d), jnp.float32) for _ in range(2 * b)]
        + [pltpu.VMEM((ch_rows, d), jnp.float32) for _ in range(b)]
        + [pltpu.SemaphoreType.DMA for _ in range(2 + 3 * b)]
    )

    @functools.partial(
        pl.kernel,
        out_type=jax.ShapeDtypeStruct((b, s, d), jnp.float32),
        mesh=mesh,
        scratch_types=scratch,
    )
    def sc_add(x_hbm, emb_hbm, out_hbm, *bufs):
        ev = bufs[0:2]
        xv = bufs[2:2 + 2 * b]
        ov = bufs[2 + 2 * b:2 + 3 * b]
        esem = bufs[2 + 3 * b:4 + 3 * b]
        xsem = bufs[4 + 3 * b:4 + 5 * b]
        osem = bufs[4 + 5 * b:4 + 6 * b]

        wid = lax.axis_index("s") * _NC + lax.axis_index("c")
        base = wid * rows_w

        def row(c):
            return base + c * ch_rows

        def load(c, cc, j):
            k = cc * b + j
            pltpu.async_copy(x_hbm.at[j, pl.ds(row(c), ch_rows)], xv[k], xsem[k])

        def load_wait(c, cc, j):
            k = cc * b + j
            pltpu.make_async_copy(
                x_hbm.at[j, pl.ds(row(c), ch_rows)], xv[k], xsem[k]
            ).wait()

        pltpu.async_copy(emb_hbm.at[pl.ds(row(0), ch_rows)], ev[0], esem[0])
        pltpu.async_copy(emb_hbm.at[pl.ds(row(1), ch_rows)], ev[1], esem[1])
        for cc in range(2):
            for j in range(b):
                load(cc, cc, j)

        @pl.loop(0, n_ch, step=2)
        def _chunks(c0):
            for cc in range(2):
                c = c0 + cc
                for j in range(b):
                    k = cc * b + j
                    load_wait(c, cc, j)
                    if j == 0:
                        pltpu.make_async_copy(
                            emb_hbm.at[pl.ds(row(c), ch_rows)], ev[cc], esem[cc]
                        ).wait()
                    @pl.when(c > 0)
                    def _():
                        pltpu.make_async_copy(
                            ov[j], out_hbm.at[j, pl.ds(row(c), ch_rows)], osem[j]
                        ).wait()

                    @plsc.parallel_loop(0, d, step=16)
                    def _vec(o):
                        for r in range(ch_rows):
                            ov[j][r, pl.ds(o, 16)] = (
                                xv[k][r, pl.ds(o, 16)] + ev[cc][r, pl.ds(o, 16)]
                            )

                    @pl.when(c + 2 < n_ch)
                    def _():
                        load(c + 2, cc, j)

                    pltpu.async_copy(
                        ov[j], out_hbm.at[j, pl.ds(row(c), ch_rows)], osem[j]
                    )
                @pl.when(c + 2 < n_ch)
                def _():
                    pltpu.async_copy(
                        emb_hbm.at[pl.ds(row(c + 2), ch_rows)], ev[cc], esem[cc]
                    )

        for j in range(b):
            pltpu.make_async_copy(
                ov[j], out_hbm.at[j, pl.ds(row(n_ch - 1), ch_rows)], osem[j]
            ).wait()

    return sc_add(x, pe)
